# baseline (speedup 1.0000x reference)
import jax
import jax.numpy as jnp
from jax import lax
from jax.experimental import pallas as pl
from jax.experimental.pallas import tpu as pltpu

B, S, D, E = 4, 8192, 768, 64
R = D // 4

NSPLIT = 2
CHUNK = 2048
NSTEPS = (B * S) // (CHUNK * NSPLIT)
CHUNKS_PER_BATCH = S // (CHUNK * NSPLIT)

_INV_SQRT2 = 0.7071067811865476


def _router_kernel(*refs):
    x_refs = refs[:NSPLIT]
    (w_ref, s_ref, c_ref, w2t_ref, b2s_ref, wrt_ref, br_ref,
     aux_ref, idx_ref, acc_ref) = refs[NSPLIT:]
    i = pl.program_id(0)
    b = i // CHUNKS_PER_BATCH

    cs = jnp.zeros((1, R), dtype=jnp.float32)
    for x_ref in x_refs:
        xb = x_ref[:]
        mu = jnp.sum(xb, axis=1, keepdims=True) * (1.0 / D)
        sq = jnp.sum(xb * xb, axis=1, keepdims=True) * (1.0 / D)
        inv = lax.rsqrt(sq - mu * mu + 1e-5)
        p = jnp.dot(xb, w_ref[:], preferred_element_type=jnp.float32)
        h1 = inv * (p - mu * s_ref[:]) + c_ref[:]
        g = 0.5 * h1 * (1.0 + lax.erf(h1 * _INV_SQRT2))
        cs = cs + jnp.sum(g, axis=0, keepdims=True)

    @pl.when(i == 0)
    def _init():
        acc_ref[:] = jnp.zeros_like(acc_ref)

    rows = lax.broadcasted_iota(jnp.int32, (B, R), 0)
    acc_ref[:] += jnp.where(rows == b, cs, 0.0)

    @pl.when(i == NSTEPS - 1)
    def _tail():
        red = acc_ref[:]
        br2 = jnp.dot(red, w2t_ref[:],
                      preferred_element_type=jnp.float32) + b2s_ref[:]
        logits = jnp.dot(br2, wrt_ref[:],
                         preferred_element_type=jnp.float32) + br_ref[:]
        m = jnp.max(logits, axis=1, keepdims=True)
        cols = lax.broadcasted_iota(jnp.int32, (B, E), 1)
        idx = jnp.min(jnp.where(logits == m, cols, E), axis=1,
                      keepdims=True)
        t = (cols == idx).astype(jnp.float32)
        bce = (jnp.maximum(logits, 0.0) - logits * t
               + jnp.log1p(jnp.exp(-jnp.abs(logits))))
        aux_ref[:] = jnp.sum(bce, axis=(0, 1), keepdims=True) * (1.0 / (B * E))
        counts = jnp.sum(t, axis=0, keepdims=True)
        m2 = jnp.max(counts, axis=1, keepdims=True)
        ecols = lax.broadcasted_iota(jnp.int32, (1, E), 1)
        idx_ref[:] = jnp.min(jnp.where(counts == m2, ecols, E), axis=1,
                             keepdims=True)


def _x_spec(j):
    return pl.BlockSpec((CHUNK, D), lambda i, j=j: (i * NSPLIT + j, 0))


@jax.jit
def kernel(hidden_states, ln_g, ln_b, w1, b1, w2, b2, wr, br):
    xf = hidden_states.reshape(B * S, D)
    wg = (w1 * ln_g).T
    s = (ln_g @ w1.T).reshape(1, R)
    c = (ln_b @ w1.T + b1).reshape(1, R)
    w2t = w2.T
    b2s = (S * b2).reshape(1, R)
    wrt = wr.T
    brr = br.reshape(1, E)

    full = lambda shape: pl.BlockSpec(shape, lambda i: (0, 0))
    aux, idx = pl.pallas_call(
        _router_kernel,
        grid=(NSTEPS,),
        in_specs=[_x_spec(j) for j in range(NSPLIT)] + [
            full((D, R)), full((1, R)), full((1, R)), full((R, R)),
            full((1, R)), full((R, E)), full((1, E)),
        ],
        out_specs=[full((1, 1)), full((1, 1))],
        out_shape=[
            jax.ShapeDtypeStruct((1, 1), jnp.float32),
            jax.ShapeDtypeStruct((1, 1), jnp.int32),
        ],
        scratch_shapes=[pltpu.VMEM((B, R), jnp.float32)],
    )(*([xf] * NSPLIT), wg, s, c, w2t, b2s, wrt, brr)
    return aux[0, 0], idx[0, 0]


# speedup vs baseline: 1.0081x; 1.0081x over previous
import jax
import jax.numpy as jnp
from jax import lax
from jax.experimental import pallas as pl
from jax.experimental.pallas import tpu as pltpu

B, S, D, E = 4, 8192, 768, 64
R = D // 4

NSPLIT = 4
CHUNK = 1024
NSTEPS = (B * S) // (CHUNK * NSPLIT)
CHUNKS_PER_BATCH = S // (CHUNK * NSPLIT)

_INV_SQRT2 = 0.7071067811865476


def _router_kernel(*refs):
    x_refs = refs[:NSPLIT]
    (w_ref, s_ref, c_ref, w2t_ref, b2s_ref, wrt_ref, br_ref,
     aux_ref, idx_ref, acc_ref) = refs[NSPLIT:]
    i = pl.program_id(0)
    b = i // CHUNKS_PER_BATCH

    cs = jnp.zeros((1, R), dtype=jnp.float32)
    for x_ref in x_refs:
        xb = x_ref[:]
        mu = jnp.sum(xb, axis=1, keepdims=True) * (1.0 / D)
        sq = jnp.sum(xb * xb, axis=1, keepdims=True) * (1.0 / D)
        inv = lax.rsqrt(sq - mu * mu + 1e-5)
        p = jnp.dot(xb, w_ref[:], preferred_element_type=jnp.float32)
        h1 = inv * (p - mu * s_ref[:]) + c_ref[:]
        g = 0.5 * h1 * (1.0 + lax.erf(h1 * _INV_SQRT2))
        cs = cs + jnp.sum(g, axis=0, keepdims=True)

    @pl.when(i == 0)
    def _init():
        acc_ref[:] = jnp.zeros_like(acc_ref)

    rows = lax.broadcasted_iota(jnp.int32, (B, R), 0)
    acc_ref[:] += jnp.where(rows == b, cs, 0.0)

    @pl.when(i == NSTEPS - 1)
    def _tail():
        red = acc_ref[:]
        br2 = jnp.dot(red, w2t_ref[:],
                      preferred_element_type=jnp.float32) + b2s_ref[:]
        logits = jnp.dot(br2, wrt_ref[:],
                         preferred_element_type=jnp.float32) + br_ref[:]
        m = jnp.max(logits, axis=1, keepdims=True)
        cols = lax.broadcasted_iota(jnp.int32, (B, E), 1)
        idx = jnp.min(jnp.where(logits == m, cols, E), axis=1,
                      keepdims=True)
        t = (cols == idx).astype(jnp.float32)
        bce = (jnp.maximum(logits, 0.0) - logits * t
               + jnp.log1p(jnp.exp(-jnp.abs(logits))))
        aux_ref[:] = jnp.sum(bce, axis=(0, 1), keepdims=True) * (1.0 / (B * E))
        counts = jnp.sum(t, axis=0, keepdims=True)
        m2 = jnp.max(counts, axis=1, keepdims=True)
        ecols = lax.broadcasted_iota(jnp.int32, (1, E), 1)
        idx_ref[:] = jnp.min(jnp.where(counts == m2, ecols, E), axis=1,
                             keepdims=True)


def _x_spec(j):
    return pl.BlockSpec((CHUNK, D), lambda i, j=j: (i * NSPLIT + j, 0))


@jax.jit
def kernel(hidden_states, ln_g, ln_b, w1, b1, w2, b2, wr, br):
    xf = hidden_states.reshape(B * S, D)
    wg = (w1 * ln_g).T
    s = (ln_g @ w1.T).reshape(1, R)
    c = (ln_b @ w1.T + b1).reshape(1, R)
    w2t = w2.T
    b2s = (S * b2).reshape(1, R)
    wrt = wr.T
    brr = br.reshape(1, E)

    full = lambda shape: pl.BlockSpec(shape, lambda i: (0, 0))
    aux, idx = pl.pallas_call(
        _router_kernel,
        grid=(NSTEPS,),
        in_specs=[_x_spec(j) for j in range(NSPLIT)] + [
            full((D, R)), full((1, R)), full((1, R)), full((R, R)),
            full((1, R)), full((R, E)), full((1, E)),
        ],
        out_specs=[full((1, 1)), full((1, 1))],
        out_shape=[
            jax.ShapeDtypeStruct((1, 1), jnp.float32),
            jax.ShapeDtypeStruct((1, 1), jnp.int32),
        ],
        scratch_shapes=[pltpu.VMEM((B, R), jnp.float32)],
    )(*([xf] * NSPLIT), wg, s, c, w2t, b2s, wrt, brr)
    return aux[0, 0], idx[0, 0]


# speedup vs baseline: 1.0950x; 1.0862x over previous
import jax
import jax.numpy as jnp
from jax import lax
from jax.experimental import pallas as pl
from jax.experimental.pallas import tpu as pltpu

B, S, D, E = 4, 8192, 768, 64
R = D // 4
RPAD = 256

CHUNK = 4096
NSTEPS = (B * S) // CHUNK
CHUNKS_PER_BATCH = S // CHUNK

_INV_SQRT2 = 0.7071067811865476


def _router_kernel(x_ref, w_ref, c_ref, w2t_ref, b2s_ref, wrt_ref, br_ref,
                   aux_ref, idx_ref, acc_ref):
    i = pl.program_id(0)
    b = i // CHUNKS_PER_BATCH

    xb = x_ref[:]
    sq = jnp.sum(xb * xb, axis=1, keepdims=True) * (1.0 / D)
    pa = jnp.dot(xb, w_ref[:], preferred_element_type=jnp.float32)
    mu = pa[:, R:R + 1]
    inv = lax.rsqrt(sq - mu * mu + 1e-5)
    h1 = inv * pa[:, :R] + c_ref[:]
    g = 0.5 * h1 * (1.0 + lax.erf(h1 * _INV_SQRT2))
    cs = jnp.sum(g, axis=0, keepdims=True)

    @pl.when(i == 0)
    def _init():
        acc_ref[:] = jnp.zeros_like(acc_ref)

    rows = lax.broadcasted_iota(jnp.int32, (B, R), 0)
    acc_ref[:] += jnp.where(rows == b, cs, 0.0)

    @pl.when(i == NSTEPS - 1)
    def _tail():
        red = acc_ref[:]
        br2 = jnp.dot(red, w2t_ref[:],
                      preferred_element_type=jnp.float32) + b2s_ref[:]
        logits = jnp.dot(br2, wrt_ref[:],
                         preferred_element_type=jnp.float32) + br_ref[:]
        m = jnp.max(logits, axis=1, keepdims=True)
        cols = lax.broadcasted_iota(jnp.int32, (B, E), 1)
        idx = jnp.min(jnp.where(logits == m, cols, E), axis=1,
                      keepdims=True)
        t = (cols == idx).astype(jnp.float32)
        bce = (jnp.maximum(logits, 0.0) - logits * t
               + jnp.log1p(jnp.exp(-jnp.abs(logits))))
        aux_ref[:] = jnp.sum(bce, axis=(0, 1), keepdims=True) * (1.0 / (B * E))
        counts = jnp.sum(t, axis=0, keepdims=True)
        m2 = jnp.max(counts, axis=1, keepdims=True)
        ecols = lax.broadcasted_iota(jnp.int32, (1, E), 1)
        idx_ref[:] = jnp.min(jnp.where(counts == m2, ecols, E), axis=1,
                             keepdims=True)


@jax.jit
def kernel(hidden_states, ln_g, ln_b, w1, b1, w2, b2, wr, br):
    xf = hidden_states.reshape(B * S, D)
    wg = (w1 * ln_g).T
    s = ln_g @ w1.T
    wc = wg - jnp.full((D, 1), 1.0 / D) * s[None, :]
    wa = jnp.concatenate(
        [wc, jnp.full((D, 1), 1.0 / D, jnp.float32),
         jnp.zeros((D, RPAD - R - 1), jnp.float32)], axis=1)
    c = (ln_b @ w1.T + b1).reshape(1, R)
    w2t = w2.T
    b2s = (S * b2).reshape(1, R)
    wrt = wr.T
    brr = br.reshape(1, E)

    full = lambda shape: pl.BlockSpec(shape, lambda i: (0, 0))
    aux, idx = pl.pallas_call(
        _router_kernel,
        grid=(NSTEPS,),
        in_specs=[
            pl.BlockSpec((CHUNK, D), lambda i: (i, 0)),
            full((D, RPAD)), full((1, R)), full((R, R)),
            full((1, R)), full((R, E)), full((1, E)),
        ],
        out_specs=[full((1, 1)), full((1, 1))],
        out_shape=[
            jax.ShapeDtypeStruct((1, 1), jnp.float32),
            jax.ShapeDtypeStruct((1, 1), jnp.int32),
        ],
        scratch_shapes=[pltpu.VMEM((B, R), jnp.float32)],
    )(xf, wa, c, w2t, b2s, wrt, brr)
    return aux[0, 0], idx[0, 0]
